# Initial kernel scaffold; baseline (speedup 1.0000x reference)
import jax
import jax.numpy as jnp
from jax.experimental import pallas as pl


def kernel(indices, weight):
    raise NotImplementedError("write your pallas kernel here")


# speedup vs baseline: 2.8083x; 2.8083x over previous
import functools

import jax
import jax.numpy as jnp
from jax import lax
from jax.experimental import pallas as pl
from jax.experimental.pallas import tpu as pltpu
from jax.experimental.pallas import tpu_sc as plsc

_VOCAB = 1000000
_D = 64
_B = 16384
_H = 50

_NC = 2
_NS = 16
_NW = _NC * _NS
_BAGS_PER_W = _B // _NW
_BAGS_PER_UNIT = 2
_IDX_PER_UNIT = _BAGS_PER_UNIT * _H
_UNITS = _BAGS_PER_W // _BAGS_PER_UNIT
_NBUF = 4
_GROUPS = _UNITS // _NBUF
_NLANE = 16
_DREG = _D // _NLANE


def _accumulate_bag(rows_ref, out_ref, row_base, out_row):
    init = tuple(
        rows_ref[row_base, pl.ds(dd * _NLANE, _NLANE)] for dd in range(_DREG)
    )

    def body(j, accs):
        r = row_base + 1 + j
        return tuple(
            accs[dd] + rows_ref[r, pl.ds(dd * _NLANE, _NLANE)]
            for dd in range(_DREG)
        )

    accs = lax.fori_loop(0, _H - 1, body, init, unroll=7)
    scale = jnp.float32(1.0 / _H)
    for dd in range(_DREG):
        out_ref[out_row, pl.ds(dd * _NLANE, _NLANE)] = jnp.maximum(
            accs[dd] * scale, 0.0
        )


def _bag_body(idx_hbm, w_hbm, out_hbm, idx_v, rows_bufs, out_bufs,
              gather_sems, store_sems):
    wid = lax.axis_index("s") * _NC + lax.axis_index("c")
    base_bag = wid * _BAGS_PER_W

    pltpu.sync_copy(idx_hbm.at[wid], idx_v)

    for b in range(_NBUF):
        pltpu.async_copy(w_hbm.at[idx_v.at[b]], rows_bufs[b], gather_sems[b])

    @pl.loop(0, _GROUPS)
    def _(g):
        for b in range(_NBUF):
            u = g * _NBUF + b
            pltpu.make_async_copy(
                w_hbm.at[idx_v.at[u]], rows_bufs[b], gather_sems[b]
            ).wait()
            @pl.when(g > 0)
            def _():
                pltpu.make_async_copy(
                    out_bufs[b],
                    out_hbm.at[pl.ds(base_bag, _BAGS_PER_UNIT)],
                    store_sems[b],
                ).wait()

            for k in range(_BAGS_PER_UNIT):
                _accumulate_bag(rows_bufs[b], out_bufs[b], k * _H, k)

            pltpu.async_copy(
                out_bufs[b],
                out_hbm.at[
                    pl.ds(base_bag + u * _BAGS_PER_UNIT, _BAGS_PER_UNIT)
                ],
                store_sems[b],
            )

            @pl.when(u + _NBUF < _UNITS)
            def _():
                pltpu.async_copy(
                    w_hbm.at[idx_v.at[u + _NBUF]], rows_bufs[b],
                    gather_sems[b],
                )

    for b in range(_NBUF):
        pltpu.make_async_copy(
            out_bufs[b],
            out_hbm.at[pl.ds(base_bag, _BAGS_PER_UNIT)],
            store_sems[b],
        ).wait()


@jax.jit
def _bag_mean_relu(idx, weight):
    mesh = plsc.VectorSubcoreMesh(core_axis_name="c", subcore_axis_name="s")
    f = pl.kernel(
        _bag_body,
        out_type=jax.ShapeDtypeStruct((_B, _D), jnp.float32),
        mesh=mesh,
        scratch_types=[
            pltpu.VMEM((_UNITS, _IDX_PER_UNIT), jnp.int32),
            [pltpu.VMEM((_IDX_PER_UNIT, _D), jnp.float32)
             for _ in range(_NBUF)],
            [pltpu.VMEM((_BAGS_PER_UNIT, _D), jnp.float32)
             for _ in range(_NBUF)],
            [pltpu.SemaphoreType.DMA for _ in range(_NBUF)],
            [pltpu.SemaphoreType.DMA for _ in range(_NBUF)],
        ],
        compiler_params=pltpu.CompilerParams(use_tc_tiling_on_sc=False),
    )
    return f(idx, weight)


def kernel(indices, weight):
    idx = indices.astype(jnp.int32).reshape(_NW, _UNITS, _IDX_PER_UNIT)
    return _bag_mean_relu(idx, weight)
